# baseline (device time: 18794 ns/iter reference)
import jax
import jax.numpy as jnp
from jax import lax
from jax.experimental import pallas as pl
from jax.experimental.pallas import tpu as pltpu

N_Y = 4

_ORDERS = {0: (1, 2, 3), 1: (0, 2, 3), 2: (1, 3, 0), 3: (2, 1, 0)}


def kernel(x):
    m_per, n = x.shape
    half = m_per // 2

    def body(x_ref, out_ref, ycomm, xcomm, ysend, yrecv, xsend, xrecv):
        my_x = lax.axis_index("x")
        my_y = lax.axis_index("y")
        my_z = lax.axis_index("z")
        other_x = 1 - my_x

        barrier_sem = pltpu.get_barrier_semaphore()
        for yp in range(N_Y):
            @pl.when(yp != my_y)
            def _():
                pl.semaphore_signal(
                    barrier_sem, inc=1,
                    device_id=(my_x, yp, my_z),
                    device_id_type=pl.DeviceIdType.MESH,
                )
        pl.semaphore_signal(
            barrier_sem, inc=1,
            device_id=(other_x, my_y, my_z),
            device_id_type=pl.DeviceIdType.MESH,
        )
        pl.semaphore_wait(barrier_sem, N_Y)

        my_half = x_ref.at[pl.ds(my_x * half, half)]

        for k in range(N_Y):
            @pl.when(my_y == k)
            def _(k=k):
                order = _ORDERS[k]

                y_sends = []
                for yp in order:
                    s = pltpu.make_async_remote_copy(
                        src_ref=my_half,
                        dst_ref=ycomm.at[k],
                        send_sem=ysend.at[yp],
                        recv_sem=yrecv.at[k, yp],
                        device_id=(my_x, yp, my_z),
                        device_id_type=pl.DeviceIdType.MESH,
                    )
                    s.start()
                    y_sends.append(s)

                out_ref[pl.ds(k * m_per, m_per), :] = x_ref[:, :]

                x_fwds = []
                for yp in order:
                    recv = pltpu.make_async_remote_copy(
                        src_ref=my_half,
                        dst_ref=ycomm.at[yp],
                        send_sem=ysend.at[yp],
                        recv_sem=yrecv.at[yp, k],
                        device_id=(my_x, yp, my_z),
                        device_id_type=pl.DeviceIdType.MESH,
                    )
                    recv.wait_recv()
                    f = pltpu.make_async_remote_copy(
                        src_ref=ycomm.at[yp],
                        dst_ref=xcomm.at[yp],
                        send_sem=xsend.at[yp],
                        recv_sem=xrecv.at[yp],
                        device_id=(other_x, k, my_z),
                        device_id_type=pl.DeviceIdType.MESH,
                    )
                    f.start()
                    x_fwds.append(f)
                    out_ref[pl.ds(yp * m_per + my_x * half, half), :] = ycomm[yp, :, :]

                for yp in order:
                    xr = pltpu.make_async_remote_copy(
                        src_ref=ycomm.at[yp],
                        dst_ref=xcomm.at[yp],
                        send_sem=xsend.at[yp],
                        recv_sem=xrecv.at[yp],
                        device_id=(other_x, k, my_z),
                        device_id_type=pl.DeviceIdType.MESH,
                    )
                    xr.wait_recv()
                    out_ref[pl.ds(yp * m_per + other_x * half, half), :] = xcomm[yp, :, :]

                for s in y_sends:
                    s.wait_send()
                for f in x_fwds:
                    f.wait_send()

    return pl.pallas_call(
        body,
        out_shape=jax.ShapeDtypeStruct((N_Y * m_per, n), x.dtype),
        in_specs=[pl.BlockSpec(memory_space=pltpu.VMEM)],
        out_specs=pl.BlockSpec(memory_space=pltpu.VMEM),
        scratch_shapes=[
            pltpu.VMEM((N_Y, half, n), x.dtype),
            pltpu.VMEM((N_Y, half, n), x.dtype),
            pltpu.SemaphoreType.DMA((N_Y,)),
            pltpu.SemaphoreType.DMA((N_Y, N_Y)),
            pltpu.SemaphoreType.DMA((N_Y,)),
            pltpu.SemaphoreType.DMA((N_Y,)),
        ],
        compiler_params=pltpu.CompilerParams(collective_id=0),
    )(x)
